# baseline (speedup 1.0000x reference)
import functools

import jax
import jax.numpy as jnp
from jax import lax
from jax.experimental import pallas as pl
from jax.experimental.pallas import tpu as pltpu
from jax.experimental.pallas import tpu_sc as plsc

NUM_ROWS = 1001
EMB = 64
OUT = 64
PAD_V = 1024
PAD_O = 64

BATCH = 4096
HIST = 200
BTOT = BATCH * HIST

NC = 2
NS = 16
NW = NC * NS
ROWS_W = BATCH // NW
SPLIT = (96, 104)
NSTEP = ROWS_W * 2
HIST_P = 256

NBUF = 8
PDIST = 4


def _proj_body(tab_ref, w_ref, b_ref, out_ref):
    r = lax.broadcasted_iota(jnp.int32, (PAD_V, 1), 0)
    tab = jnp.where(r == 0, 0.0, tab_ref[...])
    acc = lax.dot_general(tab, w_ref[...], (((1,), (1,)), ((), ())),
                          preferred_element_type=jnp.float32)
    out_ref[...] = jnp.tanh(acc + b_ref[...])


def _project_table(tab_padded, W, b2):
    return pl.pallas_call(
        _proj_body,
        out_shape=jax.ShapeDtypeStruct((PAD_V, PAD_O), jnp.float32),
    )(tab_padded, W, b2)


def _gather_body(idx_hbm, p_hbm, out_hbm, idx_v, rows_v, p_sh, gsem, osem):
    c = lax.axis_index("c")
    s = lax.axis_index("s")
    wid = s * NC + c

    @pl.when(s == 0)
    def _():
        pltpu.sync_copy(p_hbm, p_sh)

    pltpu.sync_copy(idx_hbm.at[pl.ds(wid * ROWS_W, ROWS_W)], idx_v)
    plsc.subcore_barrier()

    def parts(step, j):
        half = j % 2
        r = lax.div(step, 2)
        off = 0 if half == 0 else SPLIT[0]
        n = SPLIT[half]
        brow = wid * ROWS_W + r
        return r, off, n, brow

    def fire_gather(step, j, b):
        r, off, n, brow = parts(step, j)
        pltpu.async_copy(p_sh.at[idx_v.at[r, pl.ds(off, n)]],
                         rows_v.at[b, pl.ds(0, n)], gsem.at[b])

    def wait_gather(step, j, b):
        r, off, n, brow = parts(step, j)
        pltpu.make_async_copy(p_sh.at[idx_v.at[r, pl.ds(off, n)]],
                              rows_v.at[b, pl.ds(0, n)], gsem.at[b]).wait()

    def fire_store(step, j, b):
        r, off, n, brow = parts(step, j)
        pltpu.async_copy(rows_v.at[b, pl.ds(0, n)],
                         out_hbm.at[brow, pl.ds(off, n), pl.ds(0, OUT)],
                         osem.at[b])

    def wait_store(step, j, b):
        r, off, n, brow = parts(step, j)
        pltpu.make_async_copy(rows_v.at[b, pl.ds(0, n)],
                              out_hbm.at[brow, pl.ds(off, n), pl.ds(0, OUT)],
                              osem.at[b]).wait()

    for b in range(PDIST):
        fire_gather(b, b, b)

    def round_body(rr, carry):
        t = rr * NBUF
        for j in range(NBUF):
            step = t + j
            wait_gather(step, j, j)
            fire_store(step, j, j)
            nstep = step + PDIST
            nj = (j + PDIST) % NBUF

            @pl.when(nstep < NSTEP)
            def _():
                @pl.when(nstep >= NBUF)
                def _():
                    wait_store(nstep - NBUF, (j + PDIST) % NBUF, nj)
                fire_gather(nstep, (j + PDIST) % NBUF, nj)
        return carry

    lax.fori_loop(0, NSTEP // NBUF, round_body, 0)

    for j in range(NBUF):
        step = NSTEP - NBUF + j
        wait_store(step, j, j)


def _gather(sentiment, P):
    mesh = plsc.VectorSubcoreMesh(core_axis_name="c", subcore_axis_name="s")
    return pl.kernel(
        _gather_body,
        mesh=mesh,
        compiler_params=pltpu.CompilerParams(use_tc_tiling_on_sc=False),
        out_type=jax.ShapeDtypeStruct((BATCH, HIST, PAD_O), jnp.float32),
        scratch_types=[
            pltpu.VMEM((ROWS_W, HIST_P), jnp.int32),
            pltpu.VMEM((NBUF, SPLIT[1], PAD_O), jnp.float32),
            pltpu.MemorySpace.VMEM_SHARED((PAD_V, PAD_O), jnp.float32),
            pltpu.SemaphoreType.DMA((NBUF,)),
            pltpu.SemaphoreType.DMA((NBUF,)),
        ],
    )(sentiment, P)


def kernel(sentiment, table, W, b):
    sent_p = jnp.pad(sentiment, ((0, 0), (0, HIST_P - HIST)))
    tab_p = jnp.zeros((PAD_V, EMB), table.dtype).at[:NUM_ROWS].set(table)
    Wp = jnp.zeros((PAD_O, EMB), W.dtype).at[:OUT].set(W)
    b2 = jnp.zeros((1, PAD_O), b.dtype).at[0, :OUT].set(b)
    P = _project_table(tab_p, Wp, b2)
    return _gather(sent_p, P)[:, :, :OUT]


# speedup vs baseline: 2.0673x; 2.0673x over previous
import functools

import jax
import jax.numpy as jnp
from jax import lax
from jax.experimental import pallas as pl
from jax.experimental.pallas import tpu as pltpu
from jax.experimental.pallas import tpu_sc as plsc

NUM_ROWS = 1001
EMB = 64
OUT = 64
PAD_V = 1024
PAD_O = 64
OUT_P = 128

BATCH = 4096
HIST = 200
BTOT = BATCH * HIST

NC = 2
NS = 16
NW = NC * NS
ROWS_W = BATCH // NW
SPLIT = (96, 104)
NSTEP = ROWS_W * 2
HIST_P = 256

NBUF = 8
PDIST = 4


def _proj_body(tab_ref, w_ref, b_ref, out_ref):
    r = lax.broadcasted_iota(jnp.int32, (PAD_V, 1), 0)
    tab = jnp.where(r == 0, 0.0, tab_ref[...])
    acc = lax.dot_general(tab, w_ref[...], (((1,), (1,)), ((), ())),
                          preferred_element_type=jnp.float32)
    out_ref[...] = jnp.tanh(acc + b_ref[...])


def _project_table(tab_padded, W, b2):
    return pl.pallas_call(
        _proj_body,
        out_shape=jax.ShapeDtypeStruct((PAD_V, PAD_O), jnp.float32),
    )(tab_padded, W, b2)


def _gather_body(idx_hbm, p_hbm, out_hbm, idx_v, rows_v, p_sh, gsem, osem):
    c = lax.axis_index("c")
    s = lax.axis_index("s")
    wid = s * NC + c

    @pl.when(s == 0)
    def _():
        pltpu.sync_copy(p_hbm, p_sh)

    pltpu.sync_copy(idx_hbm.at[pl.ds(wid * ROWS_W, ROWS_W)], idx_v)
    plsc.subcore_barrier()

    def parts(step, j):
        half = j % 2
        r = lax.div(step, 2)
        off = 0 if half == 0 else SPLIT[0]
        n = SPLIT[half]
        brow = wid * ROWS_W + r
        return r, off, n, brow

    def fire_gather(step, j, b):
        r, off, n, brow = parts(step, j)
        pltpu.async_copy(p_sh.at[idx_v.at[r, pl.ds(off, n)]],
                         rows_v.at[b, pl.ds(0, n)], gsem.at[b])

    def wait_gather(step, j, b):
        r, off, n, brow = parts(step, j)
        pltpu.make_async_copy(p_sh.at[idx_v.at[r, pl.ds(off, n)]],
                              rows_v.at[b, pl.ds(0, n)], gsem.at[b]).wait()

    def fire_store(step, j, b):
        r, off, n, brow = parts(step, j)
        pltpu.async_copy(rows_v.at[b, pl.ds(0, n)],
                         out_hbm.at[brow, pl.ds(off, n), pl.ds(0, OUT)],
                         osem.at[b])

    def wait_store(step, j, b):
        r, off, n, brow = parts(step, j)
        pltpu.make_async_copy(rows_v.at[b, pl.ds(0, n)],
                              out_hbm.at[brow, pl.ds(off, n), pl.ds(0, OUT)],
                              osem.at[b]).wait()

    for b in range(PDIST):
        fire_gather(b, b, b)

    def round_body(rr, carry):
        t = rr * NBUF
        for j in range(NBUF):
            step = t + j
            wait_gather(step, j, j)
            fire_store(step, j, j)
            nstep = step + PDIST
            nj = (j + PDIST) % NBUF

            @pl.when(nstep < NSTEP)
            def _():
                @pl.when(nstep >= NBUF)
                def _():
                    wait_store(nstep - NBUF, (j + PDIST) % NBUF, nj)
                fire_gather(nstep, (j + PDIST) % NBUF, nj)
        return carry

    lax.fori_loop(0, NSTEP // NBUF, round_body, 0)

    for j in range(NBUF):
        step = NSTEP - NBUF + j
        wait_store(step, j, j)


def _gather(sentiment, P):
    mesh = plsc.VectorSubcoreMesh(core_axis_name="c", subcore_axis_name="s")
    return pl.kernel(
        _gather_body,
        mesh=mesh,
        compiler_params=pltpu.CompilerParams(use_tc_tiling_on_sc=False),
        out_type=jax.ShapeDtypeStruct((BATCH, HIST, OUT_P), jnp.float32),
        scratch_types=[
            pltpu.VMEM((ROWS_W, HIST_P), jnp.int32),
            pltpu.VMEM((NBUF, SPLIT[1], PAD_O), jnp.float32),
            pltpu.MemorySpace.VMEM_SHARED((PAD_V, PAD_O), jnp.float32),
            pltpu.SemaphoreType.DMA((NBUF,)),
            pltpu.SemaphoreType.DMA((NBUF,)),
        ],
    )(sentiment, P)


def kernel(sentiment, table, W, b):
    sent_p = jnp.pad(sentiment, ((0, 0), (0, HIST_P - HIST)))
    tab_p = jnp.zeros((PAD_V, EMB), table.dtype).at[:NUM_ROWS].set(table)
    Wp = jnp.zeros((PAD_O, EMB), W.dtype).at[:OUT].set(W)
    b2 = jnp.zeros((1, PAD_O), b.dtype).at[0, :OUT].set(b)
    P = _project_table(tab_p, Wp, b2)
    return _gather(sent_p, P)[:, :, :OUT]


# speedup vs baseline: 2.0706x; 1.0016x over previous
import functools

import jax
import jax.numpy as jnp
from jax import lax
from jax.experimental import pallas as pl
from jax.experimental.pallas import tpu as pltpu
from jax.experimental.pallas import tpu_sc as plsc

NUM_ROWS = 1001
EMB = 64
OUT = 64
PAD_V = 1024
PAD_O = 64
OUT_P = 128

BATCH = 4096
HIST = 200
BTOT = BATCH * HIST

NC = 2
NS = 16
NW = NC * NS
ROWS_W = BATCH // NW
SPLIT = (128, 72)
NSTEP = ROWS_W * 2
HIST_P = 256

NBUF = 8
PDIST = 4


def _proj_body(tab_ref, w_ref, b_ref, out_ref):
    r = lax.broadcasted_iota(jnp.int32, (PAD_V, 1), 0)
    tab = jnp.where(r == 0, 0.0, tab_ref[...])
    acc = lax.dot_general(tab, w_ref[...], (((1,), (1,)), ((), ())),
                          preferred_element_type=jnp.float32)
    out_ref[...] = jnp.tanh(acc + b_ref[...])


def _project_table(tab_padded, W, b2):
    return pl.pallas_call(
        _proj_body,
        out_shape=jax.ShapeDtypeStruct((PAD_V, PAD_O), jnp.float32),
    )(tab_padded, W, b2)


def _gather_body(idx_hbm, p_hbm, out_hbm, idx_v, rows_v, p_sh, gsem, osem):
    c = lax.axis_index("c")
    s = lax.axis_index("s")
    wid = s * NC + c

    @pl.when(s == 0)
    def _():
        pltpu.sync_copy(p_hbm, p_sh)

    pltpu.sync_copy(idx_hbm.at[pl.ds(wid * ROWS_W, ROWS_W)], idx_v)
    plsc.subcore_barrier()

    def parts(step, j):
        half = j % 2
        r = lax.div(step, 2)
        off = 0 if half == 0 else SPLIT[0]
        n = SPLIT[half]
        brow = wid * ROWS_W + r
        return r, off, n, brow

    def fire_gather(step, j, b):
        r, off, n, brow = parts(step, j)
        pltpu.async_copy(p_sh.at[idx_v.at[r, pl.ds(off, n)]],
                         rows_v.at[b, pl.ds(0, n)], gsem.at[b])

    def wait_gather(step, j, b):
        r, off, n, brow = parts(step, j)
        pltpu.make_async_copy(p_sh.at[idx_v.at[r, pl.ds(off, n)]],
                              rows_v.at[b, pl.ds(0, n)], gsem.at[b]).wait()

    def fire_store(step, j, b):
        r, off, n, brow = parts(step, j)
        pltpu.async_copy(rows_v.at[b, pl.ds(0, n)],
                         out_hbm.at[brow, pl.ds(off, n), pl.ds(0, OUT)],
                         osem.at[b])

    def wait_store(step, j, b):
        r, off, n, brow = parts(step, j)
        pltpu.make_async_copy(rows_v.at[b, pl.ds(0, n)],
                              out_hbm.at[brow, pl.ds(off, n), pl.ds(0, OUT)],
                              osem.at[b]).wait()

    for b in range(PDIST):
        fire_gather(b, b, b)

    def round_body(rr, carry):
        t = rr * NBUF
        for j in range(NBUF):
            step = t + j
            wait_gather(step, j, j)
            fire_store(step, j, j)
            nstep = step + PDIST
            nj = (j + PDIST) % NBUF

            @pl.when(nstep < NSTEP)
            def _():
                @pl.when(nstep >= NBUF)
                def _():
                    wait_store(nstep - NBUF, (j + PDIST) % NBUF, nj)
                fire_gather(nstep, (j + PDIST) % NBUF, nj)
        return carry

    lax.fori_loop(0, NSTEP // NBUF, round_body, 0)

    for j in range(NBUF):
        step = NSTEP - NBUF + j
        wait_store(step, j, j)


def _gather(sentiment, P):
    mesh = plsc.VectorSubcoreMesh(core_axis_name="c", subcore_axis_name="s")
    return pl.kernel(
        _gather_body,
        mesh=mesh,
        compiler_params=pltpu.CompilerParams(use_tc_tiling_on_sc=False),
        out_type=jax.ShapeDtypeStruct((BATCH, HIST, OUT_P), jnp.float32),
        scratch_types=[
            pltpu.VMEM((ROWS_W, HIST_P), jnp.int32),
            pltpu.VMEM((NBUF, max(SPLIT), PAD_O), jnp.float32),
            pltpu.MemorySpace.VMEM_SHARED((PAD_V, PAD_O), jnp.float32),
            pltpu.SemaphoreType.DMA((NBUF,)),
            pltpu.SemaphoreType.DMA((NBUF,)),
        ],
    )(sentiment, P)


def kernel(sentiment, table, W, b):
    sent_p = jnp.pad(sentiment, ((0, 0), (0, HIST_P - HIST)))
    tab_p = jnp.zeros((PAD_V, EMB), table.dtype).at[:NUM_ROWS].set(table)
    Wp = jnp.zeros((PAD_O, EMB), W.dtype).at[:OUT].set(W)
    b2 = jnp.zeros((1, PAD_O), b.dtype).at[0, :OUT].set(b)
    P = _project_table(tab_p, Wp, b2)
    return _gather(sent_p, P)[:, :, :OUT]


# speedup vs baseline: 2.0739x; 1.0016x over previous
import jax
import jax.numpy as jnp
from jax import lax
from jax.experimental import pallas as pl
from jax.experimental.pallas import tpu as pltpu
from jax.experimental.pallas import tpu_sc as plsc

NUM_ROWS = 1001
EMB = 64
OUT = 64
PAD_V = 1024
PAD_O = 64
OUT_P = 128

BATCH = 4096
HIST = 200
BTOT = BATCH * HIST

NC = 2
NS = 16
NW = NC * NS
ROWS_W = BATCH // NW
SPLIT = (128, 72)
NSTEP = ROWS_W * 2
HIST_P = 256

NBUF = 8
PDIST = 4


def _proj_body(tab_ref, w_ref, b_ref, out_ref):
    r = lax.broadcasted_iota(jnp.int32, (PAD_V, 1), 0)
    tab = jnp.where(r == 0, 0.0, tab_ref[...])
    acc = lax.dot_general(tab, w_ref[...], (((1,), (1,)), ((), ())),
                          preferred_element_type=jnp.float32)
    out_ref[...] = jnp.tanh(acc + b_ref[...])


def _project_table(tab_padded, W, b2):
    return pl.pallas_call(
        _proj_body,
        out_shape=jax.ShapeDtypeStruct((PAD_V, PAD_O), jnp.float32),
    )(tab_padded, W, b2)


def _gather_body(idx_hbm, p_hbm, out_hbm, idx_v, rows_v, p_sh, gsem, osem):
    c = lax.axis_index("c")
    s = lax.axis_index("s")
    wid = s * NC + c

    @pl.when(s == 0)
    def _():
        pltpu.sync_copy(p_hbm, p_sh)

    pltpu.sync_copy(idx_hbm.at[pl.ds(wid * ROWS_W, ROWS_W)], idx_v)
    plsc.subcore_barrier()

    def parts(step, j):
        half = j % 2
        r = lax.div(step, 2)
        off = 0 if half == 0 else SPLIT[0]
        n = SPLIT[half]
        brow = wid * ROWS_W + r
        return r, off, n, brow

    def fire_gather(step, j, b):
        r, off, n, brow = parts(step, j)
        pltpu.async_copy(p_sh.at[idx_v.at[r, pl.ds(off, n)]],
                         rows_v.at[b, pl.ds(0, n)], gsem.at[b])

    def wait_gather(step, j, b):
        r, off, n, brow = parts(step, j)
        pltpu.make_async_copy(p_sh.at[idx_v.at[r, pl.ds(off, n)]],
                              rows_v.at[b, pl.ds(0, n)], gsem.at[b]).wait()

    def fire_store(step, j, b):
        r, off, n, brow = parts(step, j)
        pltpu.async_copy(rows_v.at[b, pl.ds(0, n)],
                         out_hbm.at[brow, pl.ds(off, n), pl.ds(0, OUT)],
                         osem.at[b])

    def wait_store(step, j, b):
        r, off, n, brow = parts(step, j)
        pltpu.make_async_copy(rows_v.at[b, pl.ds(0, n)],
                              out_hbm.at[brow, pl.ds(off, n), pl.ds(0, OUT)],
                              osem.at[b]).wait()

    for b in range(PDIST):
        fire_gather(b, b, b)

    def round_body(rr, carry):
        t = rr * NBUF
        for j in range(NBUF):
            step = t + j
            wait_gather(step, j, j)
            fire_store(step, j, j)
            nstep = step + PDIST
            nj = (j + PDIST) % NBUF

            @pl.when(nstep < NSTEP)
            def _():
                @pl.when(nstep >= NBUF)
                def _():
                    wait_store(nstep - NBUF, (j + PDIST) % NBUF, nj)
                fire_gather(nstep, (j + PDIST) % NBUF, nj)
        return carry

    lax.fori_loop(0, NSTEP // NBUF, round_body, 0)

    for j in range(NBUF):
        step = NSTEP - NBUF + j
        wait_store(step, j, j)


def _gather(sentiment, P):
    mesh = plsc.VectorSubcoreMesh(core_axis_name="c", subcore_axis_name="s")
    return pl.kernel(
        _gather_body,
        mesh=mesh,
        compiler_params=pltpu.CompilerParams(use_tc_tiling_on_sc=False),
        out_type=jax.ShapeDtypeStruct((BATCH, HIST, OUT_P), jnp.float32),
        scratch_types=[
            pltpu.VMEM((ROWS_W, HIST_P), jnp.int32),
            pltpu.VMEM((NBUF, max(SPLIT), PAD_O), jnp.float32),
            pltpu.MemorySpace.VMEM_SHARED((PAD_V, PAD_O), jnp.float32),
            pltpu.SemaphoreType.DMA((NBUF,)),
            pltpu.SemaphoreType.DMA((NBUF,)),
        ],
    )(sentiment, P)


def kernel(sentiment, table, W, b):
    sent_p = jnp.pad(sentiment, ((0, 0), (0, HIST_P - HIST)))
    tab_p = jnp.zeros((PAD_V, EMB), table.dtype).at[:NUM_ROWS].set(table)
    Wp = jnp.zeros((PAD_O, EMB), W.dtype).at[:OUT].set(W)
    b2 = jnp.zeros((1, PAD_O), b.dtype).at[0, :OUT].set(b)
    P = _project_table(tab_p, Wp, b2)
    return _gather(sent_p, P)[:, :, :OUT]
